# baseline (speedup 1.0000x reference)
import jax
import jax.numpy as jnp
from jax import lax
from jax.experimental import pallas as pl
from jax.experimental.pallas import tpu as pltpu
from jax.experimental.pallas import tpu_sc as plsc

_NC, _NS = 1, 16
_NW = _NC * _NS
_CHUNK_ROWS = 16
_NBUF = 4
_NBUF_SH = 3


def _sc_copy_body(emb_hbm, out_hbm, bufs, shbufs, sems_in, sems_out,
                  sems_shin, sems_shout):
    seq, dim = out_hbm.shape
    rows_per_w = seq // _NW
    n_chunks = rows_per_w // _CHUNK_ROWS
    wid = lax.axis_index("s") * _NC + lax.axis_index("c")
    sid = lax.axis_index("s")
    base = wid * rows_per_w

    in_copies = [None] * n_chunks
    out_copies = [None] * n_chunks

    def start_in(i):
        off = base + i * _CHUNK_ROWS
        if i % 2 == 0:
            b = (i // 2) % _NBUF
            c = pltpu.make_async_copy(
                emb_hbm.at[pl.ds(off, _CHUNK_ROWS)], bufs.at[b],
                sems_in.at[b])
        else:
            b = (i // 2) % _NBUF_SH
            c = pltpu.make_async_copy(
                emb_hbm.at[pl.ds(off, _CHUNK_ROWS)], shbufs.at[sid, b],
                sems_shin.at[b])
        c.start()
        in_copies[i] = c

    def start_out(i):
        off = base + i * _CHUNK_ROWS
        if i % 2 == 0:
            b = (i // 2) % _NBUF
            c = pltpu.make_async_copy(
                bufs.at[b], out_hbm.at[pl.ds(off, _CHUNK_ROWS)],
                sems_out.at[b])
        else:
            b = (i // 2) % _NBUF_SH
            c = pltpu.make_async_copy(
                shbufs.at[sid, b], out_hbm.at[pl.ds(off, _CHUNK_ROWS)],
                sems_shout.at[b])
        c.start()
        out_copies[i] = c

    waited = [False] * n_chunks
    for i in range(n_chunks + 1):
        if i < n_chunks:
            j = i - 2 * (_NBUF if i % 2 == 0 else _NBUF_SH)
            if j >= 0:
                out_copies[j].wait()
                waited[j] = True
            start_in(i)
        if i >= 1:
            in_copies[i - 1].wait()
            start_out(i - 1)
    for i in range(n_chunks):
        if not waited[i]:
            out_copies[i].wait()


def kernel(x, emb):
    seq, dim = x.shape[1], emb.shape[1]
    mesh = plsc.VectorSubcoreMesh(core_axis_name="c", subcore_axis_name="s", num_cores=1)
    k = pl.kernel(
        _sc_copy_body,
        out_type=jax.ShapeDtypeStruct((seq, dim), emb.dtype),
        mesh=mesh,
        scratch_types=[
            pltpu.VMEM((_NBUF, _CHUNK_ROWS, dim), emb.dtype),
            pltpu.VMEM_SHARED((_NS, _NBUF_SH, _CHUNK_ROWS, dim), emb.dtype),
            pltpu.SemaphoreType.DMA((_NBUF,)),
            pltpu.SemaphoreType.DMA((_NBUF,)),
            pltpu.SemaphoreType.DMA((_NBUF_SH,)),
            pltpu.SemaphoreType.DMA((_NBUF_SH,)),
        ],
    )
    return k(emb)


# speedup vs baseline: 1.2103x; 1.2103x over previous
import jax
import jax.numpy as jnp
from jax import lax
from jax.experimental import pallas as pl
from jax.experimental.pallas import tpu as pltpu
from jax.experimental.pallas import tpu_sc as plsc

_NC, _NS = 2, 16
_NW = _NC * _NS
_CHUNK_ROWS = 16
_NBUF = 4
_NBUF_SH = 3


def _sc_copy_body(emb_hbm, out_hbm, bufs, shbufs, sems_in, sems_out,
                  sems_shin, sems_shout):
    seq, dim = out_hbm.shape
    rows_per_w = seq // _NW
    n_chunks = rows_per_w // _CHUNK_ROWS
    wid = lax.axis_index("s") * _NC + lax.axis_index("c")
    sid = lax.axis_index("s")
    base = wid * rows_per_w

    in_copies = [None] * n_chunks
    out_copies = [None] * n_chunks

    def start_in(i):
        off = base + i * _CHUNK_ROWS
        if i % 2 == 0:
            b = (i // 2) % _NBUF
            c = pltpu.make_async_copy(
                emb_hbm.at[pl.ds(off, _CHUNK_ROWS)], bufs.at[b],
                sems_in.at[b])
        else:
            b = (i // 2) % _NBUF_SH
            c = pltpu.make_async_copy(
                emb_hbm.at[pl.ds(off, _CHUNK_ROWS)], shbufs.at[sid, b],
                sems_shin.at[b])
        c.start()
        in_copies[i] = c

    def start_out(i):
        off = base + i * _CHUNK_ROWS
        if i % 2 == 0:
            b = (i // 2) % _NBUF
            c = pltpu.make_async_copy(
                bufs.at[b], out_hbm.at[pl.ds(off, _CHUNK_ROWS)],
                sems_out.at[b])
        else:
            b = (i // 2) % _NBUF_SH
            c = pltpu.make_async_copy(
                shbufs.at[sid, b], out_hbm.at[pl.ds(off, _CHUNK_ROWS)],
                sems_shout.at[b])
        c.start()
        out_copies[i] = c

    waited = [False] * n_chunks
    for i in range(n_chunks + 1):
        if i < n_chunks:
            j = i - 2 * (_NBUF if i % 2 == 0 else _NBUF_SH)
            if j >= 0:
                out_copies[j].wait()
                waited[j] = True
            start_in(i)
        if i >= 1:
            in_copies[i - 1].wait()
            start_out(i - 1)
    for i in range(n_chunks):
        if not waited[i]:
            out_copies[i].wait()


def kernel(x, emb):
    seq, dim = x.shape[1], emb.shape[1]
    mesh = plsc.VectorSubcoreMesh(core_axis_name="c", subcore_axis_name="s")
    k = pl.kernel(
        _sc_copy_body,
        out_type=jax.ShapeDtypeStruct((seq, dim), emb.dtype),
        mesh=mesh,
        scratch_types=[
            pltpu.VMEM((_NBUF, _CHUNK_ROWS, dim), emb.dtype),
            pltpu.VMEM_SHARED((_NS, _NBUF_SH, _CHUNK_ROWS, dim), emb.dtype),
            pltpu.SemaphoreType.DMA((_NBUF,)),
            pltpu.SemaphoreType.DMA((_NBUF,)),
            pltpu.SemaphoreType.DMA((_NBUF_SH,)),
            pltpu.SemaphoreType.DMA((_NBUF_SH,)),
        ],
    )
    return k(emb)
